# Initial kernel scaffold; baseline (speedup 1.0000x reference)
import jax
import jax.numpy as jnp
from jax.experimental import pallas as pl


def kernel(m1, m2, m3, polar, indices):
    raise NotImplementedError("write your pallas kernel here")


# speedup vs baseline: 36.3479x; 36.3479x over previous
import functools

import jax
import jax.numpy as jnp
from jax import lax
from jax.experimental import pallas as pl
from jax.experimental.pallas import tpu as pltpu
from jax.experimental.pallas import tpu_sc as plsc

_NC, _NS, _L = 2, 16, 16
_NW = _NC * _NS
_C = 1024
_K = _C // 128
_G = _C // _L


def _make_sc_kernel(E):
    n_chunks = E // _C
    nb = E // 128
    mesh = plsc.VectorSubcoreMesh(core_axis_name="c", subcore_axis_name="s")

    @functools.partial(
        pl.kernel,
        out_type=jax.ShapeDtypeStruct((4, E), jnp.float32),
        mesh=mesh,
        scratch_types=[
            pltpu.VMEM((_K, 128), jnp.int32),
            pltpu.VMEM((_K, 128), jnp.int32),
            pltpu.VMEM((_C, 16), jnp.float32),
            pltpu.VMEM((_C, 16), jnp.float32),
            pltpu.VMEM((4, _C), jnp.float32),
            pltpu.SemaphoreType.DMA,
        ],
        compiler_params=pltpu.CompilerParams(
            needs_layout_passes=False, use_tc_tiling_on_sc=False),
    )
    def kern(table_hbm, ind_hbm, out_hbm, idx1_v, idx2_v, rows1_v, rows2_v,
             out_v, sem):
        wid = lax.axis_index("s") * _NC + lax.axis_index("c")
        n_base = n_chunks // _NW
        n_extra = n_chunks % _NW
        my_n = n_base + jnp.where(wid < n_extra, 1, 0)
        lane = lax.iota(jnp.int32, _L)

        def chunk_body(i, _):
            ci = wid + i * _NW
            base = ci * _C
            pltpu.sync_copy(ind_hbm.at[0, pl.ds(ci * _K, _K), :], idx1_v)
            pltpu.sync_copy(ind_hbm.at[1, pl.ds(ci * _K, _K), :], idx2_v)
            copies = []
            for j in range(_K):
                copies.append(pltpu.async_copy(
                    table_hbm.at[idx1_v.at[j]],
                    rows1_v.at[pl.ds(j * 128, 128), :], sem))
                copies.append(pltpu.async_copy(
                    table_hbm.at[idx2_v.at[j]],
                    rows2_v.at[pl.ds(j * 128, 128), :], sem))
            for cp in copies:
                cp.wait()

            def group_body(g, _):
                row = lane + g * _L
                for b in range(4):
                    cols = [jnp.full((_L,), c, jnp.int32)
                            for c in (b, 4 + b, 8 + b, 12 + b)]
                    m1_1 = plsc.load_gather(rows1_v, [row, cols[0]])
                    m2_1 = plsc.load_gather(rows1_v, [row, cols[1]])
                    m3_1 = plsc.load_gather(rows1_v, [row, cols[2]])
                    p_1 = plsc.load_gather(rows1_v, [row, cols[3]])
                    m1_2 = plsc.load_gather(rows2_v, [row, cols[0]])
                    m2_2 = plsc.load_gather(rows2_v, [row, cols[1]])
                    m3_2 = plsc.load_gather(rows2_v, [row, cols[2]])
                    p_2 = plsc.load_gather(rows2_v, [row, cols[3]])
                    num = m1_1 * m3_2 + m3_1 * m1_2 + 2.1 * (m2_1 * m2_2)
                    den = m1_1 / p_1 + m1_2 / p_2
                    out_v[b, pl.ds(g * _L, _L)] = 2.0 * num / den
                return _

            lax.fori_loop(0, _G, group_body, 0, unroll=2)
            pltpu.sync_copy(out_v, out_hbm.at[:, pl.ds(base, _C)])
            return _

        lax.fori_loop(0, my_n, chunk_body, 0)

    return kern


def kernel(m1, m2, m3, polar, indices):
    E = indices.shape[1]
    table = jnp.concatenate([m1, m2, m3, polar], axis=0).T
    ind3 = indices.reshape(2, E // 128, 128)
    return _make_sc_kernel(E)(table, ind3)


# speedup vs baseline: 40.8054x; 1.1226x over previous
import functools

import jax
import jax.numpy as jnp
from jax import lax
from jax.experimental import pallas as pl
from jax.experimental.pallas import tpu as pltpu
from jax.experimental.pallas import tpu_sc as plsc

_NC, _NS, _L = 2, 16, 16
_NW = _NC * _NS
_C = 1024
_K = _C // 128
_G = _C // _L


def _make_sc_kernel(E):
    w_edges = E // _NW
    cpw = (w_edges + _C - 1) // _C
    n_pairs = (cpw + 1) // 2
    last_off = w_edges - _C
    mesh = plsc.VectorSubcoreMesh(core_axis_name="c", subcore_axis_name="s")

    @functools.partial(
        pl.kernel,
        out_type=jax.ShapeDtypeStruct((4, E), jnp.float32),
        mesh=mesh,
        scratch_types=[
            pltpu.VMEM((2, _C), jnp.int32),
            pltpu.VMEM((2, _C), jnp.int32),
            pltpu.VMEM((2, _C, 16), jnp.float32),
            pltpu.VMEM((2, _C, 16), jnp.float32),
            pltpu.VMEM((4, _C), jnp.float32),
            pltpu.SemaphoreType.DMA,
            pltpu.SemaphoreType.DMA,
        ],
        compiler_params=pltpu.CompilerParams(
            needs_layout_passes=False, use_tc_tiling_on_sc=False),
    )
    def kern(table_hbm, ind_hbm, out_hbm, idx1_v, idx2_v, rows1_v, rows2_v,
             out_v, sem0, sem1):
        wid = lax.axis_index("s") * _NC + lax.axis_index("c")
        wbase = wid * w_edges
        lane = lax.iota(jnp.int32, _L)
        sems = (sem0, sem1)

        def base_of(c):
            return wbase + jnp.minimum(c * _C, last_off)

        def gather_copies(p, make_only=False):
            mk = pltpu.make_async_copy if make_only else None
            cps = []
            for j in range(_K):
                s = pl.ds(j * 128, 128)
                for idx_v, rows_v in ((idx1_v, rows1_v), (idx2_v, rows2_v)):
                    src = table_hbm.at[idx_v.at[p, s]]
                    dst = rows_v.at[p, s, :]
                    if make_only:
                        cps.append(mk(src, dst, sems[p]))
                    else:
                        cps.append(pltpu.async_copy(src, dst, sems[p]))
            return cps

        def fire(p, base):
            pltpu.sync_copy(ind_hbm.at[0, pl.ds(base, _C)], idx1_v.at[p])
            pltpu.sync_copy(ind_hbm.at[1, pl.ds(base, _C)], idx2_v.at[p])
            gather_copies(p)

        def consume(p, base):
            for cp in gather_copies(p, make_only=True):
                cp.wait()
            r1 = rows1_v.at[p]
            r2 = rows2_v.at[p]

            def group_body(g, _):
                row = lane + g * _L
                for b in range(4):
                    cols = [jnp.full((_L,), c, jnp.int32)
                            for c in (b, 4 + b, 8 + b, 12 + b)]
                    m1_1 = plsc.load_gather(r1, [row, cols[0]])
                    m2_1 = plsc.load_gather(r1, [row, cols[1]])
                    m3_1 = plsc.load_gather(r1, [row, cols[2]])
                    p_1 = plsc.load_gather(r1, [row, cols[3]])
                    m1_2 = plsc.load_gather(r2, [row, cols[0]])
                    m2_2 = plsc.load_gather(r2, [row, cols[1]])
                    m3_2 = plsc.load_gather(r2, [row, cols[2]])
                    p_2 = plsc.load_gather(r2, [row, cols[3]])
                    num = m1_1 * m3_2 + m3_1 * m1_2 + 2.1 * (m2_1 * m2_2)
                    den = m1_1 / p_1 + m1_2 / p_2
                    out_v[b, pl.ds(g * _L, _L)] = 2.0 * num / den
                return _

            lax.fori_loop(0, _G, group_body, 0, unroll=2)
            pltpu.sync_copy(out_v, out_hbm.at[:, pl.ds(base, _C)])

        fire(0, base_of(0))

        def pair_body(j, _):
            c0 = 2 * j
            fire(1, base_of(c0 + 1))
            consume(0, base_of(c0))

            @pl.when(j < n_pairs - 1)
            def _fire_next():
                fire(0, base_of(c0 + 2))

            consume(1, base_of(c0 + 1))
            return _

        lax.fori_loop(0, n_pairs, pair_body, 0)

    return kern


def kernel(m1, m2, m3, polar, indices):
    E = indices.shape[1]
    table = jnp.concatenate([m1, m2, m3, polar], axis=0).T
    return _make_sc_kernel(E)(table, indices)


# speedup vs baseline: 43.3067x; 1.0613x over previous
import functools

import jax
import jax.numpy as jnp
from jax import lax
from jax.experimental import pallas as pl
from jax.experimental.pallas import tpu as pltpu
from jax.experimental.pallas import tpu_sc as plsc

_NC, _NS, _L = 2, 16, 16
_NW = _NC * _NS
_C = 1024
_K = _C // 128
_G = _C // _L


def _make_sc_kernel(E):
    w_edges = E // _NW
    cpw = (w_edges + _C - 1) // _C
    n_pairs = (cpw + 1) // 2
    last_off = w_edges - _C
    mesh = plsc.VectorSubcoreMesh(core_axis_name="c", subcore_axis_name="s")

    @functools.partial(
        pl.kernel,
        out_type=jax.ShapeDtypeStruct((4, E), jnp.float32),
        mesh=mesh,
        scratch_types=[
            pltpu.VMEM((2, _C), jnp.int32),
            pltpu.VMEM((2, _C), jnp.int32),
            pltpu.VMEM((2, _C, 16), jnp.float32),
            pltpu.VMEM((2, _C, 16), jnp.float32),
            pltpu.VMEM((4, _C), jnp.float32),
            pltpu.SemaphoreType.DMA,
            pltpu.SemaphoreType.DMA,
        ],
        compiler_params=pltpu.CompilerParams(
            needs_layout_passes=False, use_tc_tiling_on_sc=False),
    )
    def kern(table_hbm, ind_hbm, out_hbm, idx1_v, idx2_v, rows1_v, rows2_v,
             out_v, sem0, sem1):
        wid = lax.axis_index("s") * _NC + lax.axis_index("c")
        wbase = wid * w_edges
        lane = lax.iota(jnp.int32, _L)
        sems = (sem0, sem1)

        def base_of(c):
            return wbase + jnp.minimum(c * _C, last_off)

        def gather_copies(p, make_only=False):
            mk = pltpu.make_async_copy if make_only else None
            cps = []
            for j in range(_K):
                s = pl.ds(j * 128, 128)
                for idx_v, rows_v in ((idx1_v, rows1_v), (idx2_v, rows2_v)):
                    src = table_hbm.at[idx_v.at[p, s]]
                    dst = rows_v.at[p, s, :]
                    if make_only:
                        cps.append(mk(src, dst, sems[p]))
                    else:
                        cps.append(pltpu.async_copy(src, dst, sems[p]))
            return cps

        def fire(p, base):
            pltpu.sync_copy(ind_hbm.at[0, pl.ds(base, _C)], idx1_v.at[p])
            pltpu.sync_copy(ind_hbm.at[1, pl.ds(base, _C)], idx2_v.at[p])
            gather_copies(p)

        def consume(p, base):
            for cp in gather_copies(p, make_only=True):
                cp.wait()
            r1 = rows1_v.at[p]
            r2 = rows2_v.at[p]

            def group_body(g, _):
                row = lane + g * _L
                for b in range(4):
                    cols = [jnp.full((_L,), c, jnp.int32)
                            for c in (b, 4 + b, 8 + b, 12 + b)]
                    m1_1 = plsc.load_gather(r1, [row, cols[0]])
                    m2_1 = plsc.load_gather(r1, [row, cols[1]])
                    m3_1 = plsc.load_gather(r1, [row, cols[2]])
                    p_1 = plsc.load_gather(r1, [row, cols[3]])
                    m1_2 = plsc.load_gather(r2, [row, cols[0]])
                    m2_2 = plsc.load_gather(r2, [row, cols[1]])
                    m3_2 = plsc.load_gather(r2, [row, cols[2]])
                    p_2 = plsc.load_gather(r2, [row, cols[3]])
                    num = m1_1 * m3_2 + m3_1 * m1_2 + 2.1 * (m2_1 * m2_2)
                    t = num * (p_1 * p_2)
                    den = m1_1 * p_2 + m1_2 * p_1
                    out_v[b, pl.ds(g * _L, _L)] = (t + t) / den
                return _

            lax.fori_loop(0, _G, group_body, 0, unroll=2)
            pltpu.sync_copy(out_v, out_hbm.at[:, pl.ds(base, _C)])

        fire(0, base_of(0))

        def pair_body(j, _):
            c0 = 2 * j
            fire(1, base_of(c0 + 1))
            consume(0, base_of(c0))

            @pl.when(j < n_pairs - 1)
            def _fire_next():
                fire(0, base_of(c0 + 2))

            consume(1, base_of(c0 + 1))
            return _

        lax.fori_loop(0, n_pairs, pair_body, 0)

    return kern


def kernel(m1, m2, m3, polar, indices):
    E = indices.shape[1]
    table = jnp.concatenate([m1, m2, m3, polar], axis=0).T
    return _make_sc_kernel(E)(table, indices)


# speedup vs baseline: 46.8343x; 1.0815x over previous
import functools

import jax
import jax.numpy as jnp
from jax import lax
from jax.experimental import pallas as pl
from jax.experimental.pallas import tpu as pltpu
from jax.experimental.pallas import tpu_sc as plsc

_NC, _NS, _L = 2, 16, 16
_C = 1024
_K = _C // 128
_G = _C // _L


def _make_sc_kernel(E, N):
    w_edges = E // _NS
    cpw = (w_edges + _C - 1) // _C
    n_pairs = (cpw + 1) // 2
    last_off = w_edges - _C
    rows_per_sub = N // _NS
    mesh = plsc.VectorSubcoreMesh(core_axis_name="c", subcore_axis_name="s")

    @functools.partial(
        pl.kernel,
        out_type=jax.ShapeDtypeStruct((4, E), jnp.float32),
        mesh=mesh,
        scratch_types=[
            pltpu.VMEM((2, _C), jnp.int32),
            pltpu.VMEM((2, _C), jnp.int32),
            pltpu.VMEM((2, _C, 8), jnp.float32),
            pltpu.VMEM((2, _C, 8), jnp.float32),
            pltpu.VMEM((2, _C), jnp.float32),
            pltpu.VMEM_SHARED((N, 8), jnp.float32),
            pltpu.SemaphoreType.DMA,
            pltpu.SemaphoreType.DMA,
        ],
        compiler_params=pltpu.CompilerParams(
            needs_layout_passes=False, use_tc_tiling_on_sc=False),
    )
    def kern(table_hbm, ind_hbm, out_hbm, idx1_v, idx2_v, rows1_v, rows2_v,
             out_v, table_sp, sem0, sem1):
        cid = lax.axis_index("c")
        sub = lax.axis_index("s")
        wbase = sub * w_edges
        lane = lax.iota(jnp.int32, _L)
        sems = (sem0, sem1)

        seg = pl.ds(sub * rows_per_sub, rows_per_sub)
        pltpu.sync_copy(table_hbm.at[cid, seg], table_sp.at[seg])
        plsc.subcore_barrier()

        def base_of(c):
            return wbase + jnp.minimum(c * _C, last_off)

        def gather_copies(p, make_only=False):
            mk = pltpu.make_async_copy if make_only else None
            cps = []
            for j in range(_K):
                s = pl.ds(j * 128, 128)
                for idx_v, rows_v in ((idx1_v, rows1_v), (idx2_v, rows2_v)):
                    dst = rows_v.at[p, s, :]
                    if make_only:
                        cps.append(mk(table_hbm.at[0].at[idx_v.at[p, s]], dst,
                                      sems[p]))
                    else:
                        cps.append(pltpu.async_copy(
                            table_sp.at[idx_v.at[p, s]], dst, sems[p]))
            return cps

        def fire(p, base):
            pltpu.sync_copy(ind_hbm.at[0, pl.ds(base, _C)], idx1_v.at[p])
            pltpu.sync_copy(ind_hbm.at[1, pl.ds(base, _C)], idx2_v.at[p])
            gather_copies(p)

        def consume(p, base):
            for cp in gather_copies(p, make_only=True):
                cp.wait()
            r1 = rows1_v.at[p]
            r2 = rows2_v.at[p]

            def group_body(g, _):
                row = lane + g * _L
                for b in range(2):
                    cols = [jnp.full((_L,), c, jnp.int32)
                            for c in (4 * b, 4 * b + 1, 4 * b + 2, 4 * b + 3)]
                    m1_1 = plsc.load_gather(r1, [row, cols[0]])
                    m2_1 = plsc.load_gather(r1, [row, cols[1]])
                    m3_1 = plsc.load_gather(r1, [row, cols[2]])
                    p_1 = plsc.load_gather(r1, [row, cols[3]])
                    m1_2 = plsc.load_gather(r2, [row, cols[0]])
                    m2_2 = plsc.load_gather(r2, [row, cols[1]])
                    m3_2 = plsc.load_gather(r2, [row, cols[2]])
                    p_2 = plsc.load_gather(r2, [row, cols[3]])
                    num = m1_1 * m3_2 + m3_1 * m1_2 + 2.1 * (m2_1 * m2_2)
                    t = num * (p_1 * p_2)
                    den = m1_1 * p_2 + m1_2 * p_1
                    out_v[b, pl.ds(g * _L, _L)] = (t + t) / den
                return _

            lax.fori_loop(0, _G, group_body, 0, unroll=2)
            pltpu.sync_copy(
                out_v, out_hbm.at[pl.ds(2 * cid, 2), pl.ds(base, _C)])

        fire(0, base_of(0))

        def pair_body(j, _):
            c0 = 2 * j
            fire(1, base_of(c0 + 1))
            consume(0, base_of(c0))

            @pl.when(j < n_pairs - 1)
            def _fire_next():
                fire(0, base_of(c0 + 2))

            consume(1, base_of(c0 + 1))
            return _

        lax.fori_loop(0, n_pairs, pair_body, 0)

    return kern


def kernel(m1, m2, m3, polar, indices):
    E = indices.shape[1]
    B, N = m1.shape
    x = jnp.stack([m1, m2, m3, polar], axis=1)
    table = x.reshape(2, 2, 4, N).transpose(0, 3, 1, 2).reshape(2, N, 8)
    return _make_sc_kernel(E, N)(table, indices)


# speedup vs baseline: 49.2761x; 1.0521x over previous
import functools

import jax
import jax.numpy as jnp
from jax import lax
from jax.experimental import pallas as pl
from jax.experimental.pallas import tpu as pltpu
from jax.experimental.pallas import tpu_sc as plsc

_NC, _NS, _L = 2, 16, 16
_C = 2048
_K = _C // 128
_G = _C // _L


def _make_sc_kernel(E, N):
    w_edges = E // _NS
    cpw = (w_edges + _C - 1) // _C
    n_pairs = (cpw + 1) // 2
    last_off = w_edges - _C
    rows_per_sub = N // _NS
    mesh = plsc.VectorSubcoreMesh(core_axis_name="c", subcore_axis_name="s")

    @functools.partial(
        pl.kernel,
        out_type=jax.ShapeDtypeStruct((4, E), jnp.float32),
        mesh=mesh,
        scratch_types=[
            pltpu.VMEM((2, _C), jnp.int32),
            pltpu.VMEM((2, _C), jnp.int32),
            pltpu.VMEM((2, _C, 8), jnp.float32),
            pltpu.VMEM((2, _C, 8), jnp.float32),
            pltpu.VMEM((2, _C), jnp.float32),
            pltpu.VMEM_SHARED((N, 8), jnp.float32),
            pltpu.SemaphoreType.DMA,
            pltpu.SemaphoreType.DMA,
        ],
        compiler_params=pltpu.CompilerParams(
            needs_layout_passes=False, use_tc_tiling_on_sc=False),
    )
    def kern(table_hbm, ind_hbm, out_hbm, idx1_v, idx2_v, rows1_v, rows2_v,
             out_v, table_sp, sem0, sem1):
        cid = lax.axis_index("c")
        sub = lax.axis_index("s")
        wbase = sub * w_edges
        lane = lax.iota(jnp.int32, _L)
        sems = (sem0, sem1)

        seg = pl.ds(sub * rows_per_sub, rows_per_sub)
        pltpu.sync_copy(table_hbm.at[cid, seg], table_sp.at[seg])
        plsc.subcore_barrier()

        def base_of(c):
            return wbase + jnp.minimum(c * _C, last_off)

        def gather_copies(p, make_only=False):
            mk = pltpu.make_async_copy if make_only else None
            cps = []
            for j in range(_K):
                s = pl.ds(j * 128, 128)
                for idx_v, rows_v in ((idx1_v, rows1_v), (idx2_v, rows2_v)):
                    dst = rows_v.at[p, s, :]
                    if make_only:
                        cps.append(mk(table_hbm.at[0].at[idx_v.at[p, s]], dst,
                                      sems[p]))
                    else:
                        cps.append(pltpu.async_copy(
                            table_sp.at[idx_v.at[p, s]], dst, sems[p]))
            return cps

        def fire(p, base):
            pltpu.sync_copy(ind_hbm.at[0, pl.ds(base, _C)], idx1_v.at[p])
            pltpu.sync_copy(ind_hbm.at[1, pl.ds(base, _C)], idx2_v.at[p])
            gather_copies(p)

        def consume(p, base):
            for cp in gather_copies(p, make_only=True):
                cp.wait()
            r1 = rows1_v.at[p]
            r2 = rows2_v.at[p]

            def group_body(g, _):
                row = lane + g * _L
                for b in range(2):
                    cols = [jnp.full((_L,), c, jnp.int32)
                            for c in (4 * b, 4 * b + 1, 4 * b + 2, 4 * b + 3)]
                    m1_1 = plsc.load_gather(r1, [row, cols[0]])
                    m2_1 = plsc.load_gather(r1, [row, cols[1]])
                    m3_1 = plsc.load_gather(r1, [row, cols[2]])
                    p_1 = plsc.load_gather(r1, [row, cols[3]])
                    m1_2 = plsc.load_gather(r2, [row, cols[0]])
                    m2_2 = plsc.load_gather(r2, [row, cols[1]])
                    m3_2 = plsc.load_gather(r2, [row, cols[2]])
                    p_2 = plsc.load_gather(r2, [row, cols[3]])
                    num = m1_1 * m3_2 + m3_1 * m1_2 + 2.1 * (m2_1 * m2_2)
                    t = num * (p_1 * p_2)
                    den = m1_1 * p_2 + m1_2 * p_1
                    out_v[b, pl.ds(g * _L, _L)] = (t + t) / den
                return _

            lax.fori_loop(0, _G, group_body, 0, unroll=2)
            pltpu.sync_copy(
                out_v, out_hbm.at[pl.ds(2 * cid, 2), pl.ds(base, _C)])

        fire(0, base_of(0))

        def pair_body(j, _):
            c0 = 2 * j
            fire(1, base_of(c0 + 1))
            consume(0, base_of(c0))

            @pl.when(j < n_pairs - 1)
            def _fire_next():
                fire(0, base_of(c0 + 2))

            consume(1, base_of(c0 + 1))
            return _

        lax.fori_loop(0, n_pairs, pair_body, 0)

    return kern


def kernel(m1, m2, m3, polar, indices):
    E = indices.shape[1]
    B, N = m1.shape
    x = jnp.stack([m1, m2, m3, polar], axis=1)
    table = x.reshape(2, 2, 4, N).transpose(0, 3, 1, 2).reshape(2, N, 8)
    return _make_sc_kernel(E, N)(table, indices)


# speedup vs baseline: 49.5843x; 1.0063x over previous
import functools

import jax
import jax.numpy as jnp
from jax import lax
from jax.experimental import pallas as pl
from jax.experimental.pallas import tpu as pltpu
from jax.experimental.pallas import tpu_sc as plsc

_NC, _NS, _L = 2, 16, 16
_C = 2048
_K = _C // 128
_G = _C // _L


def _make_sc_kernel(E, N):
    w_edges = E // _NS
    cpw = (w_edges + _C - 1) // _C
    n_pairs = (cpw + 1) // 2
    last_off = w_edges - _C
    rows_per_sub = N // _NS
    mesh = plsc.VectorSubcoreMesh(core_axis_name="c", subcore_axis_name="s")

    @functools.partial(
        pl.kernel,
        out_type=jax.ShapeDtypeStruct((4, E), jnp.float32),
        mesh=mesh,
        scratch_types=[
            pltpu.VMEM((2, _C), jnp.int32),
            pltpu.VMEM((2, _C), jnp.int32),
            pltpu.VMEM((2, _C, 8), jnp.float32),
            pltpu.VMEM((2, _C, 8), jnp.float32),
            pltpu.VMEM((2, _C), jnp.float32),
            pltpu.VMEM_SHARED((N, 8), jnp.float32),
            pltpu.SemaphoreType.DMA,
            pltpu.SemaphoreType.DMA,
        ],
        compiler_params=pltpu.CompilerParams(
            needs_layout_passes=False, use_tc_tiling_on_sc=False),
    )
    def kern(table_hbm, ind_hbm, out_hbm, idx1_v, idx2_v, rows1_v, rows2_v,
             out_v, table_sp, sem0, sem1):
        cid = lax.axis_index("c")
        sub = lax.axis_index("s")
        wbase = sub * w_edges
        lane = lax.iota(jnp.int32, _L)
        sems = (sem0, sem1)

        seg = pl.ds(sub * rows_per_sub, rows_per_sub)
        pltpu.sync_copy(table_hbm.at[cid, seg], table_sp.at[seg])
        plsc.subcore_barrier()

        def base_of(c):
            return wbase + jnp.minimum(c * _C, last_off)

        def gather_copies(p, make_only=False):
            mk = pltpu.make_async_copy if make_only else None
            cps = []
            for j in range(_K):
                s = pl.ds(j * 128, 128)
                for idx_v, rows_v in ((idx1_v, rows1_v), (idx2_v, rows2_v)):
                    dst = rows_v.at[p, s, :]
                    if make_only:
                        cps.append(mk(table_hbm.at[0].at[idx_v.at[p, s]], dst,
                                      sems[p]))
                    else:
                        cps.append(pltpu.async_copy(
                            table_sp.at[idx_v.at[p, s]], dst, sems[p]))
            return cps

        def fire(p, base):
            pltpu.sync_copy(ind_hbm.at[0, pl.ds(base, _C)], idx1_v.at[p])
            pltpu.sync_copy(ind_hbm.at[1, pl.ds(base, _C)], idx2_v.at[p])
            gather_copies(p)

        def consume(p, base):
            for cp in gather_copies(p, make_only=True):
                cp.wait()
            r1 = rows1_v.at[p]
            r2 = rows2_v.at[p]

            def group_body(g, _):
                row = lane + g * _L
                for b in range(2):
                    cols = [jnp.full((_L,), c, jnp.int32)
                            for c in (4 * b, 4 * b + 1, 4 * b + 2, 4 * b + 3)]
                    m1_1 = plsc.load_gather(r1, [row, cols[0]])
                    m2_1 = plsc.load_gather(r1, [row, cols[1]])
                    m3_1 = plsc.load_gather(r1, [row, cols[2]])
                    p_1 = plsc.load_gather(r1, [row, cols[3]])
                    m1_2 = plsc.load_gather(r2, [row, cols[0]])
                    m2_2 = plsc.load_gather(r2, [row, cols[1]])
                    m3_2 = plsc.load_gather(r2, [row, cols[2]])
                    p_2 = plsc.load_gather(r2, [row, cols[3]])
                    num = m1_1 * m3_2 + m3_1 * m1_2 + 2.1 * (m2_1 * m2_2)
                    t = num * (p_1 * p_2)
                    den = m1_1 * p_2 + m1_2 * p_1
                    out_v[b, pl.ds(g * _L, _L)] = (t + t) / den
                return _

            lax.fori_loop(0, _G, group_body, 0, unroll=4)
            pltpu.sync_copy(
                out_v, out_hbm.at[pl.ds(2 * cid, 2), pl.ds(base, _C)])

        fire(0, base_of(0))

        def pair_body(j, _):
            c0 = 2 * j
            fire(1, base_of(c0 + 1))
            consume(0, base_of(c0))

            @pl.when(j < n_pairs - 1)
            def _fire_next():
                fire(0, base_of(c0 + 2))

            consume(1, base_of(c0 + 1))
            return _

        lax.fori_loop(0, n_pairs, pair_body, 0)

    return kern


def kernel(m1, m2, m3, polar, indices):
    E = indices.shape[1]
    B, N = m1.shape
    x = jnp.stack([m1, m2, m3, polar], axis=1)
    table = x.reshape(2, 2, 4, N).transpose(0, 3, 1, 2).reshape(2, N, 8)
    return _make_sc_kernel(E, N)(table, indices)
